# baseline (speedup 1.0000x reference)
import functools

import jax
import jax.numpy as jnp
from jax import lax
from jax.experimental import pallas as pl
from jax.experimental.pallas import tpu as pltpu
from jax.experimental.pallas import tpu_sc as plsc

DIM = 128
BATCH = 16384
NC = 2
NS = 16
L = 16
NW = NC * NS
BPW = BATCH // NW
CH = 128
NCHUNK = BPW // CH
VPR = DIM // L
TPC = CH // L

_mesh = plsc.VectorSubcoreMesh(
    core_axis_name="c", subcore_axis_name="s", num_cores=NC, num_subcores=NS
)


@functools.partial(
    pl.kernel,
    out_type=jax.ShapeDtypeStruct((NW, L), jnp.float32),
    mesh=_mesh,
    scratch_types=[
        pltpu.VMEM((3 * BPW,), jnp.int32),
        pltpu.VMEM((2, 3, CH, DIM), jnp.float32),
        pltpu.VMEM((L,), jnp.float32),
        pltpu.SemaphoreType.DMA,
        pltpu.SemaphoreType.DMA,
    ],
)
def _transe_loss_sc(ins_hbm, rel_hbm, sample_hbm, out_hbm, idx_v,
                    rows_v, acc_v, sem0, sem1):
    wid = lax.axis_index("s") * NC + lax.axis_index("c")
    sems = (sem0, sem1)

    for j in range(3):
        pltpu.sync_copy(sample_hbm.at[pl.ds(j * BATCH + wid * BPW, BPW)],
                        idx_v.at[pl.ds(j * BPW, BPW)])

    def islice(j, c):
        return idx_v.at[pl.ds(j * BPW + c * CH, CH)]

    def fire(c, buf):
        sem = sems[buf]
        pltpu.async_copy(ins_hbm.at[islice(0, c)], rows_v.at[buf, 0], sem)
        pltpu.async_copy(rel_hbm.at[islice(1, c)], rows_v.at[buf, 1], sem)
        pltpu.async_copy(ins_hbm.at[islice(2, c)], rows_v.at[buf, 2], sem)

    def drain(c, buf):
        sem = sems[buf]
        pltpu.make_async_copy(ins_hbm.at[islice(0, c)],
                              rows_v.at[buf, 0], sem).wait()
        pltpu.make_async_copy(rel_hbm.at[islice(1, c)],
                              rows_v.at[buf, 1], sem).wait()
        pltpu.make_async_copy(ins_hbm.at[islice(2, c)],
                              rows_v.at[buf, 2], sem).wait()

    fire(0, 0)
    fire(1, 1)

    accs = tuple(jnp.zeros((L,), jnp.float32) for _ in range(VPR))
    for c in range(NCHUNK):
        buf = c % 2
        drain(c, buf)

        def body(s, accs):
            new = []
            for k in range(VPR):
                sl = pl.ds(k * L, L)
                e1 = rows_v[buf, 0, s, sl]
                r = rows_v[buf, 1, s, sl]
                e2 = rows_v[buf, 2, s, sl]
                d = (e1 + r) - e2
                new.append(accs[k] + d * d)
            return tuple(new)

        accs = lax.fori_loop(0, CH, body, accs)

        if c + 2 < NCHUNK:
            fire(c + 2, buf)

    total = accs[0]
    for k in range(1, VPR):
        total = total + accs[k]
    acc_v[...] = total
    pltpu.sync_copy(acc_v, out_hbm.at[wid])


def kernel(ins_emb, rel_emb, sample):
    cols = sample.astype(jnp.int32).T.reshape(-1)
    partials = _transe_loss_sc(ins_emb, rel_emb, cols)
    return jnp.sum(partials)


# speedup vs baseline: 1.0355x; 1.0355x over previous
import functools

import jax
import jax.numpy as jnp
from jax import lax
from jax.experimental import pallas as pl
from jax.experimental.pallas import tpu as pltpu
from jax.experimental.pallas import tpu_sc as plsc

DIM = 128
BATCH = 16384
NC = 2
NS = 16
L = 16
NW = NC * NS
BPW = BATCH // NW
CH = 128
NCHUNK = BPW // CH
VPR = DIM // L
TPC = CH // L

_mesh = plsc.VectorSubcoreMesh(
    core_axis_name="c", subcore_axis_name="s", num_cores=NC, num_subcores=NS
)


@functools.partial(
    pl.kernel,
    out_type=jax.ShapeDtypeStruct((NW, L), jnp.float32),
    mesh=_mesh,
    scratch_types=[
        pltpu.VMEM((3 * BPW,), jnp.int32),
        pltpu.VMEM((2, 3, CH, DIM), jnp.float32),
        pltpu.VMEM((L,), jnp.float32),
        pltpu.SemaphoreType.DMA,
        pltpu.SemaphoreType.DMA,
    ],
)
def _transe_loss_sc(ins_hbm, rel_hbm, sample_hbm, out_hbm, idx_v,
                    rows_v, acc_v, sem0, sem1):
    wid = lax.axis_index("s") * NC + lax.axis_index("c")
    sems = (sem0, sem1)

    for j in range(3):
        pltpu.async_copy(sample_hbm.at[pl.ds(j * BATCH + wid * BPW, BPW)],
                         idx_v.at[pl.ds(j * BPW, BPW)], sem0)
    for j in range(3):
        pltpu.make_async_copy(sample_hbm.at[pl.ds(j * BATCH + wid * BPW, BPW)],
                              idx_v.at[pl.ds(j * BPW, BPW)], sem0).wait()

    def islice(j, c):
        return idx_v.at[pl.ds(j * BPW + c * CH, CH)]

    def fire(c, buf):
        sem = sems[buf]
        pltpu.async_copy(ins_hbm.at[islice(0, c)], rows_v.at[buf, 0], sem)
        pltpu.async_copy(rel_hbm.at[islice(1, c)], rows_v.at[buf, 1], sem)
        pltpu.async_copy(ins_hbm.at[islice(2, c)], rows_v.at[buf, 2], sem)

    def drain(c, buf):
        sem = sems[buf]
        pltpu.make_async_copy(ins_hbm.at[islice(0, c)],
                              rows_v.at[buf, 0], sem).wait()
        pltpu.make_async_copy(rel_hbm.at[islice(1, c)],
                              rows_v.at[buf, 1], sem).wait()
        pltpu.make_async_copy(ins_hbm.at[islice(2, c)],
                              rows_v.at[buf, 2], sem).wait()

    fire(0, 0)
    fire(1, 1)

    accs = tuple(jnp.zeros((L,), jnp.float32) for _ in range(VPR))
    for c in range(NCHUNK):
        buf = c % 2
        drain(c, buf)

        def body(s, accs):
            new = []
            for k in range(VPR):
                sl = pl.ds(k * L, L)
                e1 = rows_v[buf, 0, s, sl]
                r = rows_v[buf, 1, s, sl]
                e2 = rows_v[buf, 2, s, sl]
                d = (e1 + r) - e2
                new.append(accs[k] + d * d)
            return tuple(new)

        accs = lax.fori_loop(0, CH, body, accs)

        if c + 2 < NCHUNK:
            fire(c + 2, buf)

    total = accs[0]
    for k in range(1, VPR):
        total = total + accs[k]
    acc_v[...] = total
    pltpu.sync_copy(acc_v, out_hbm.at[wid])


def kernel(ins_emb, rel_emb, sample):
    cols = sample.astype(jnp.int32).T.reshape(-1)
    partials = _transe_loss_sc(ins_emb, rel_emb, cols)
    return jnp.sum(partials)
